# baseline (speedup 1.0000x reference)
import functools

import jax
import jax.numpy as jnp
from jax import lax
from jax.experimental import pallas as pl
from jax.experimental.pallas import tpu as pltpu
from jax.experimental.pallas import tpu_sc as plsc

B, C, H, W = 4, 192, 224, 224
TILE = 16
GH, GW = H // TILE, W // TILE
K = GH * GW
KMAX_L0 = 64
THETA_ON = 0.5
GATE_TEMP = 1.0
C_HEAVY, C_CHEAP = 1.0, 0.1
BUDGET_PER_SAMPLE = 0.3 * K
MU = 1.0

TR = TILE * W
NSTEP = B * GH
NB = 5

L = 16
KPAD = 208
NCH = KPAD // L


def _sc_gate_body(logits_ref, probs_ref, gates_ref, scal_ref,
                  row_v, probs_v, skey_v, tmp_v):
    c = lax.axis_index("c")
    s = lax.axis_index("s")

    @pl.when((c == 0) & (s < B))
    def _row_work():
        pltpu.sync_copy(logits_ref.at[s], row_v)
        for i in range(NCH):
            x = row_v[pl.ds(i * L, L)]
            p = 1.0 / (1.0 + jnp.exp(-x / GATE_TEMP))
            probs_v[pl.ds(i * L, L)] = p
        pltpu.sync_copy(probs_v, probs_ref.at[s])
        for i in range(NCH):
            x = row_v[pl.ds(i * L, L)]
            p = probs_v[pl.ds(i * L, L)]
            masked = jnp.where(p >= THETA_ON, x, jnp.float32(-1e30))
            u = lax.bitcast_convert_type(masked, jnp.int32)
            skey_v[pl.ds(i * L, L)] = jnp.where(
                u < 0, u ^ jnp.int32(0x7FFFFFFF), u)
        int_min = jnp.int32(-(2**31))

        def bit_step(i, cand):
            trial = cand | (jnp.int32(1) << (jnp.int32(31) - i))
            scand = trial ^ int_min
            cnt = jnp.int32(0)
            for j in range(NCH):
                m = skey_v[pl.ds(j * L, L)] >= scand
                cnt = cnt + jnp.sum(m.astype(jnp.int32))
            return jnp.where(cnt >= KMAX_L0, trial, cand)

        kth = lax.fori_loop(0, 32, bit_step, jnp.int32(0)) ^ int_min
        for i in range(NCH):
            p = probs_v[pl.ds(i * L, L)]
            keep = (skey_v[pl.ds(i * L, L)] >= kth) & (p >= THETA_ON)
            row_v[pl.ds(i * L, L)] = keep.astype(jnp.float32)
        pltpu.sync_copy(row_v, gates_ref.at[s])

    plsc.subcore_barrier()

    @pl.when((c == 0) & (s == 0))
    def _scalars():
        lane0 = lax.broadcasted_iota(jnp.int32, (L,), 0)
        total = jnp.zeros((L,), jnp.float32)
        for b in range(B):
            pltpu.sync_copy(probs_ref.at[b], probs_v)
            for i in range(NCH):
                p = probs_v[pl.ds(i * L, L)]
                valid = (i * L + lane0) < K
                total = total + jnp.where(valid, p, 0.0)
        ec = jnp.sum(total) * (C_HEAVY - C_CHEAP) + jnp.float32(B * K * C_CHEAP)
        bl = MU * jnp.maximum(ec - jnp.float32(BUDGET_PER_SAMPLE * B), 0.0)
        tmp_v[...] = jnp.where(lane0 == 0, ec, jnp.where(lane0 == 1, bl, 0.0))
        pltpu.sync_copy(tmp_v, scal_ref)


def _sc_gate(logits_pad):
    mesh = plsc.VectorSubcoreMesh(core_axis_name="c", subcore_axis_name="s",
                                  num_cores=2, num_subcores=16)
    fn = pl.kernel(
        _sc_gate_body,
        out_type=[
            jax.ShapeDtypeStruct((B, KPAD), jnp.float32),
            jax.ShapeDtypeStruct((B, KPAD), jnp.float32),
            jax.ShapeDtypeStruct((L,), jnp.float32),
        ],
        mesh=mesh,
        compiler_params=pltpu.CompilerParams(needs_layout_passes=False),
        scratch_types=[
            pltpu.VMEM((KPAD,), jnp.float32),
            pltpu.VMEM((KPAD,), jnp.float32),
            pltpu.VMEM((KPAD,), jnp.int32),
            pltpu.VMEM((L,), jnp.float32),
        ],
    )
    return fn(logits_pad)


def _heavy_body(g_ref, x_hbm, wh_ref, wd_ref, wa_ref,
                h_hbm, d_hbm, a_hbm,
                xb, hb, db, ab, si, sh, sd, sa):
    def in_copy(step, slot):
        b, g = step // GH, step % GH
        return pltpu.make_async_copy(x_hbm.at[b, :, g], xb.at[slot],
                                     si.at[slot])

    def h_copy(step, slot):
        b, g = step // GH, step % GH
        return pltpu.make_async_copy(hb.at[slot], h_hbm.at[b, :, g],
                                     sh.at[slot])

    def d_copy(step, slot):
        b, g = step // GH, step % GH
        return pltpu.make_async_copy(db.at[slot], d_hbm.at[b, :, g],
                                     sd.at[slot])

    def a_copy(step, slot):
        b, g = step // GH, step % GH
        return pltpu.make_async_copy(ab.at[slot], a_hbm.at[b, :, g],
                                     sa.at[slot])

    for s in range(NB):
        in_copy(s, s).start()

    lane = jax.lax.broadcasted_iota(jnp.int32, (1, TR), 1)
    tidx = (lane // TILE) % GW

    def loop_body(step, carry):
        slot = lax.rem(step, NB)
        in_copy(step, slot).wait()

        @pl.when(step >= NB)
        def _wait_outs():
            h_copy(step - NB, slot).wait()
            d_copy(step - NB, slot).wait()
            a_copy(step - NB, slot).wait()

        x = xb[slot]
        grow = g_ref[step]
        pm = jnp.zeros((1, TR), jnp.float32)
        for j in range(GW):
            pm = pm + jnp.where(tidx == j, grow[:, j:j + 1], 0.0)

        dn = (((0,), (0,)), ((), ()))
        hh = jax.nn.gelu(lax.dot_general(
            wh_ref[...], x, dn, preferred_element_type=jnp.float32))
        dd = jnp.tanh(lax.dot_general(
            wd_ref[...], x, dn, preferred_element_type=jnp.float32))
        aa = jax.nn.sigmoid(lax.dot_general(
            wa_ref[...], x, (((1,), (0,)), ((), ())),
            preferred_element_type=jnp.float32))

        hb[slot] = pm * hh + (1.0 - pm) * x
        db[slot] = pm * dd
        ab[slot] = pm * aa

        h_copy(step, slot).start()
        d_copy(step, slot).start()
        a_copy(step, slot).start()

        @pl.when(step + NB < NSTEP)
        def _next_in():
            in_copy(step + NB, slot).start()

        return carry

    lax.fori_loop(0, NSTEP, loop_body, jnp.int32(0))

    for s in range(NSTEP - NB, NSTEP):
        slot = s % NB
        h_copy(s, slot).wait()
        d_copy(s, slot).wait()
        a_copy(s, slot).wait()


@jax.jit
def kernel(dense_features, utility_logits, W_heavy, W_detail, w_alpha):
    logits_pad = jnp.pad(utility_logits, ((0, 0), (0, KPAD - K)),
                         constant_values=-1e30)
    probs_pad, gates_pad, scal = _sc_gate(logits_pad)
    probs = probs_pad[:, :K]
    gates = gates_pad[:, :K]

    x4 = dense_features.reshape(B, C, GH, TR)
    g3 = gates.reshape(B * GH, 1, GW)
    wa2 = w_alpha.reshape(1, C)

    heavy4, detail4, alpha4 = pl.pallas_call(
        _heavy_body,
        in_specs=[
            pl.BlockSpec(memory_space=pltpu.VMEM),
            pl.BlockSpec(memory_space=pltpu.HBM),
            pl.BlockSpec(memory_space=pltpu.VMEM),
            pl.BlockSpec(memory_space=pltpu.VMEM),
            pl.BlockSpec(memory_space=pltpu.VMEM),
        ],
        out_specs=[
            pl.BlockSpec(memory_space=pltpu.HBM),
            pl.BlockSpec(memory_space=pltpu.HBM),
            pl.BlockSpec(memory_space=pltpu.HBM),
        ],
        out_shape=[
            jax.ShapeDtypeStruct((B, C, GH, TR), jnp.float32),
            jax.ShapeDtypeStruct((B, C, GH, TR), jnp.float32),
            jax.ShapeDtypeStruct((B, 1, GH, TR), jnp.float32),
        ],
        scratch_shapes=[
            pltpu.VMEM((NB, C, TR), jnp.float32),
            pltpu.VMEM((NB, C, TR), jnp.float32),
            pltpu.VMEM((NB, C, TR), jnp.float32),
            pltpu.VMEM((NB, 1, TR), jnp.float32),
            pltpu.SemaphoreType.DMA((NB,)),
            pltpu.SemaphoreType.DMA((NB,)),
            pltpu.SemaphoreType.DMA((NB,)),
            pltpu.SemaphoreType.DMA((NB,)),
        ],
    )(g3, x4, W_heavy, W_detail, wa2)

    heavy_features = heavy4.reshape(B, C, H, W)
    detail_map = detail4.reshape(B, C, H, W)
    alpha = alpha4.reshape(B, 1, H, W)
    return (heavy_features, detail_map, alpha, probs, gates,
            scal[0], scal[1])


# speedup vs baseline: 1.1526x; 1.1526x over previous
import functools

import jax
import jax.numpy as jnp
from jax import lax
from jax.experimental import pallas as pl
from jax.experimental.pallas import tpu as pltpu
from jax.experimental.pallas import tpu_sc as plsc

B, C, H, W = 4, 192, 224, 224
TILE = 16
GH, GW = H // TILE, W // TILE
K = GH * GW
KMAX_L0 = 64
THETA_ON = 0.5
GATE_TEMP = 1.0
C_HEAVY, C_CHEAP = 1.0, 0.1
BUDGET_PER_SAMPLE = 0.3 * K
MU = 1.0

TR = TILE * W
RB = 2

L = 16
KPAD = 208
NCH = KPAD // L


def _sc_gate_body(logits_ref, probs_ref, gates_ref, scal_ref,
                  row_v, probs_v, skey_v, tmp_v):
    c = lax.axis_index("c")
    s = lax.axis_index("s")

    @pl.when((c == 0) & (s < B))
    def _row_work():
        pltpu.sync_copy(logits_ref.at[s], row_v)
        for i in range(NCH):
            x = row_v[pl.ds(i * L, L)]
            p = 1.0 / (1.0 + jnp.exp(-x / GATE_TEMP))
            probs_v[pl.ds(i * L, L)] = p
        pltpu.sync_copy(probs_v, probs_ref.at[s])
        for i in range(NCH):
            x = row_v[pl.ds(i * L, L)]
            p = probs_v[pl.ds(i * L, L)]
            masked = jnp.where(p >= THETA_ON, x, jnp.float32(-1e30))
            u = lax.bitcast_convert_type(masked, jnp.int32)
            skey_v[pl.ds(i * L, L)] = jnp.where(
                u < 0, u ^ jnp.int32(0x7FFFFFFF), u)
        int_min = jnp.int32(-(2**31))

        def bit_step(i, cand):
            trial = cand | (jnp.int32(1) << (jnp.int32(31) - i))
            scand = trial ^ int_min
            cnt = jnp.int32(0)
            for j in range(NCH):
                m = skey_v[pl.ds(j * L, L)] >= scand
                cnt = cnt + jnp.sum(m.astype(jnp.int32))
            return jnp.where(cnt >= KMAX_L0, trial, cand)

        kth = lax.fori_loop(0, 32, bit_step, jnp.int32(0)) ^ int_min
        for i in range(NCH):
            p = probs_v[pl.ds(i * L, L)]
            keep = (skey_v[pl.ds(i * L, L)] >= kth) & (p >= THETA_ON)
            row_v[pl.ds(i * L, L)] = keep.astype(jnp.float32)
        pltpu.sync_copy(row_v, gates_ref.at[s])

    plsc.subcore_barrier()

    @pl.when((c == 0) & (s == 0))
    def _scalars():
        lane0 = lax.broadcasted_iota(jnp.int32, (L,), 0)
        total = jnp.zeros((L,), jnp.float32)
        for b in range(B):
            pltpu.sync_copy(probs_ref.at[b], probs_v)
            for i in range(NCH):
                p = probs_v[pl.ds(i * L, L)]
                valid = (i * L + lane0) < K
                total = total + jnp.where(valid, p, 0.0)
        ec = jnp.sum(total) * (C_HEAVY - C_CHEAP) + jnp.float32(B * K * C_CHEAP)
        bl = MU * jnp.maximum(ec - jnp.float32(BUDGET_PER_SAMPLE * B), 0.0)
        tmp_v[...] = jnp.where(lane0 == 0, ec, jnp.where(lane0 == 1, bl, 0.0))
        pltpu.sync_copy(tmp_v, scal_ref)


def _sc_gate(logits_pad):
    mesh = plsc.VectorSubcoreMesh(core_axis_name="c", subcore_axis_name="s",
                                  num_cores=2, num_subcores=16)
    fn = pl.kernel(
        _sc_gate_body,
        out_type=[
            jax.ShapeDtypeStruct((B, KPAD), jnp.float32),
            jax.ShapeDtypeStruct((B, KPAD), jnp.float32),
            jax.ShapeDtypeStruct((L,), jnp.float32),
        ],
        mesh=mesh,
        compiler_params=pltpu.CompilerParams(needs_layout_passes=False),
        scratch_types=[
            pltpu.VMEM((KPAD,), jnp.float32),
            pltpu.VMEM((KPAD,), jnp.float32),
            pltpu.VMEM((KPAD,), jnp.int32),
            pltpu.VMEM((L,), jnp.float32),
        ],
    )
    return fn(logits_pad)


def _heavy_kernel(g_ref, x_ref, wh_ref, wd_ref, wa_ref, h_ref, d_ref, a_ref):
    x = x_ref[0]
    grow = g_ref[0]

    lane = jax.lax.broadcasted_iota(jnp.int32, (1, RB * TR), 1)
    tidx = (lane // TR) * GW + (lane // TILE) % GW
    pm = jnp.zeros((1, RB * TR), jnp.float32)
    for j in range(RB * GW):
        pm = pm + jnp.where(tidx == j, grow[:, j:j + 1], 0.0)

    dn = (((0,), (0,)), ((), ()))
    hh = jax.nn.gelu(jax.lax.dot_general(
        wh_ref[...], x, dn, preferred_element_type=jnp.float32))
    dd = jnp.tanh(jax.lax.dot_general(
        wd_ref[...], x, dn, preferred_element_type=jnp.float32))
    aa = jax.nn.sigmoid(jax.lax.dot_general(
        wa_ref[...], x, (((1,), (0,)), ((), ())),
        preferred_element_type=jnp.float32))

    h_ref[0] = pm * hh + (1.0 - pm) * x
    d_ref[0] = pm * dd
    a_ref[0] = pm * aa


@jax.jit
def kernel(dense_features, utility_logits, W_heavy, W_detail, w_alpha):
    logits_pad = jnp.pad(utility_logits, ((0, 0), (0, KPAD - K)),
                         constant_values=-1e30)
    probs_pad, gates_pad, scal = _sc_gate(logits_pad)
    probs = probs_pad[:, :K]
    gates = gates_pad[:, :K]

    x2 = dense_features.reshape(B, C, H * W)
    g3 = gates.reshape(B * GH // RB, 1, RB * GW)
    wa2 = w_alpha.reshape(1, C)

    heavy2, detail2, alpha2 = pl.pallas_call(
        _heavy_kernel,
        grid=(B, GH // RB),
        in_specs=[
            pl.BlockSpec((1, 1, RB * GW), lambda b, g: (b * (GH // RB) + g, 0, 0)),
            pl.BlockSpec((1, C, RB * TR), lambda b, g: (b, 0, g)),
            pl.BlockSpec((C, C), lambda b, g: (0, 0)),
            pl.BlockSpec((C, C), lambda b, g: (0, 0)),
            pl.BlockSpec((1, C), lambda b, g: (0, 0)),
        ],
        out_specs=[
            pl.BlockSpec((1, C, RB * TR), lambda b, g: (b, 0, g)),
            pl.BlockSpec((1, C, RB * TR), lambda b, g: (b, 0, g)),
            pl.BlockSpec((1, 1, RB * TR), lambda b, g: (b, 0, g)),
        ],
        out_shape=[
            jax.ShapeDtypeStruct((B, C, H * W), jnp.float32),
            jax.ShapeDtypeStruct((B, C, H * W), jnp.float32),
            jax.ShapeDtypeStruct((B, 1, H * W), jnp.float32),
        ],
    )(g3, x2, W_heavy, W_detail, wa2)

    heavy_features = heavy2.reshape(B, C, H, W)
    detail_map = detail2.reshape(B, C, H, W)
    alpha = alpha2.reshape(B, 1, H, W)
    return (heavy_features, detail_map, alpha, probs, gates,
            scal[0], scal[1])
